# baseline (speedup 1.0000x reference)
import functools

import jax
import jax.numpy as jnp
from jax import lax
from jax.experimental import pallas as pl
from jax.experimental.pallas import tpu as pltpu
from jax.experimental.pallas import tpu_sc as plsc

N = 10000
K = 32
D = 128
OUT = 128
P = 16
KP_EXTENT = 0.05

E = N * K
IDX_ROWS = E // 128
NW = 32
ROWS_PER_W = IDX_ROWS // NW
TAIL_ROWS = IDX_ROWS - NW * ROWS_PER_W
GR = 3
GROUPS = ROWS_PER_W // GR
CHUNK = GR * 128

B = 200
GRID = N // B


def _sc_mesh():
    return plsc.VectorSubcoreMesh(
        core_axis_name="c", subcore_axis_name="s", num_cores=2, num_subcores=16
    )


def _worker_loop(body):
    wid = lax.axis_index("s") * 2 + lax.axis_index("c")
    wbase = wid * ROWS_PER_W

    def g_body(g, carry):
        body(wbase + g * GR, GR)
        return carry

    lax.fori_loop(0, GROUPS, g_body, 0)

    @pl.when(wid < TAIL_ROWS)
    def _tail():
        body(NW * ROWS_PER_W + wid, 1)


def _sc_gather_features(idx1d, ftab):
    @functools.partial(
        pl.kernel,
        out_type=jax.ShapeDtypeStruct((E, D), jnp.float32),
        mesh=_sc_mesh(),
        scratch_types=[
            pltpu.VMEM((CHUNK,), jnp.int32),
            pltpu.VMEM((CHUNK, D), jnp.float32),
            pltpu.SemaphoreType.DMA,
        ],
    )
    def k(idx_hbm, ftab_hbm, nx_hbm, idx_v, rows_v, sem):
        def do_chunk(r0, nrows):
            e0 = r0 * 128
            nedge = nrows * 128
            pltpu.sync_copy(idx_hbm.at[pl.ds(e0, nedge)],
                            idx_v.at[pl.ds(0, nedge)])
            descs = []
            for j in range(nrows):
                descs.append(pltpu.async_copy(
                    ftab_hbm.at[idx_v.at[pl.ds(j * 128, 128)]],
                    rows_v.at[pl.ds(j * 128, 128)], sem))
            for dsc in descs:
                dsc.wait()
            pltpu.sync_copy(rows_v.at[pl.ds(0, nedge)],
                            nx_hbm.at[pl.ds(e0, nedge)])

        _worker_loop(do_chunk)

    return k(idx1d, ftab)


def _sc_gather_positions(idx1d, ptab):
    @functools.partial(
        pl.kernel,
        out_type=(
            jax.ShapeDtypeStruct((E,), jnp.float32),
            jax.ShapeDtypeStruct((E,), jnp.float32),
            jax.ShapeDtypeStruct((E,), jnp.float32),
        ),
        mesh=_sc_mesh(),
        scratch_types=[
            pltpu.VMEM((CHUNK,), jnp.int32),
            pltpu.VMEM((CHUNK, 16), jnp.float32),
            pltpu.VMEM((CHUNK,), jnp.float32),
            pltpu.VMEM((CHUNK,), jnp.float32),
            pltpu.VMEM((CHUNK,), jnp.float32),
            pltpu.SemaphoreType.DMA,
        ],
        compiler_params=pltpu.CompilerParams(
            needs_layout_passes=False,
            use_tc_tiling_on_sc=False,
        ),
    )
    def k(idx_hbm, ptab_hbm, px_hbm, py_hbm, pz_hbm,
          idx_v, prow_v, pcx_v, pcy_v, pcz_v, sem):
        pc = (pcx_v, pcy_v, pcz_v)
        phbm = (px_hbm, py_hbm, pz_hbm)

        def do_chunk(r0, nrows):
            e0 = r0 * 128
            nedge = nrows * 128
            pltpu.sync_copy(idx_hbm.at[pl.ds(e0, nedge)],
                            idx_v.at[pl.ds(0, nedge)])
            descs = []
            for j in range(nrows):
                descs.append(pltpu.async_copy(
                    ptab_hbm.at[idx_v.at[pl.ds(j * 128, 128)]],
                    prow_v.at[pl.ds(j * 128, 128)], sem))
            for dsc in descs:
                dsc.wait()
            base_iota = lax.iota(jnp.int32, 16)

            def tr_body(g, carry):
                rows = g * 16 + base_iota
                for c in range(3):
                    v = plsc.load_gather(
                        prow_v, [rows, jnp.full((16,), c, jnp.int32)])
                    pc[c][pl.ds(g * 16, 16)] = v
                return carry

            lax.fori_loop(0, nedge // 16, tr_body, 0)
            for c in range(3):
                pltpu.sync_copy(pc[c].at[pl.ds(0, nedge)],
                                phbm[c].at[pl.ds(e0, nedge)])

        _worker_loop(do_chunk)

    return k(idx1d, ptab)


BM = 8
MB = B // BM
EM = BM * K


def _tc_body(nx_ref, px_ref, py_ref, pz_ref, q_ref, kp_ref, w_ref, o_ref):
    lane = jax.lax.broadcasted_iota(jnp.int32, (BM, EM), 1)
    row = jax.lax.broadcasted_iota(jnp.int32, (BM, EM), 0)
    mask = (lane // K == row).astype(jnp.float32)

    for m in range(MB):
        xm = nx_ref[pl.ds(m * EM, EM), :]
        pxm = px_ref[pl.ds(m * BM, BM), :]
        pym = py_ref[pl.ds(m * BM, BM), :]
        pzm = pz_ref[pl.ds(m * BM, BM), :]
        dx = pxm - q_ref[pl.ds(m * BM, BM), 0:1]
        dy = pym - q_ref[pl.ds(m * BM, BM), 1:2]
        dz = pzm - q_ref[pl.ds(m * BM, BM), 2:3]

        s = jnp.sum(xm, axis=1, keepdims=True)
        ind = (s > 0.0).astype(jnp.float32)
        cnt = jnp.dot(mask, ind, preferred_element_type=jnp.float32)

        acc = jnp.zeros((BM, OUT), jnp.float32)
        for p in range(P):
            ex = dx - kp_ref[p, 0]
            ey = dy - kp_ref[p, 1]
            ez = dz - kp_ref[p, 2]
            sq = ex * ex + ey * ey + ez * ez
            w = jnp.maximum(1.0 - jnp.sqrt(sq) * (1.0 / KP_EXTENT), 0.0)
            a = jnp.tile(w, (1, BM)) * mask
            wf = jnp.dot(a, xm, preferred_element_type=jnp.float32,
                         precision=jax.lax.Precision.HIGHEST)
            acc += jnp.dot(wf, w_ref[p], preferred_element_type=jnp.float32,
                           precision=jax.lax.Precision.HIGHEST)

        inv = 1.0 / jnp.maximum(cnt, 1.0)
        o_ref[pl.ds(m * BM, BM), :] = acc * inv


def kernel(q_pts, s_pts, x, neighbor_idxs, kernel_points, weights):
    idx1d = neighbor_idxs.astype(jnp.int32).reshape(E)
    ftab = jnp.concatenate([x, jnp.zeros((1, D), jnp.float32)], axis=0)
    ptab = jnp.pad(s_pts, ((0, 1), (0, 13)))
    kp_pad = jnp.pad(kernel_points, ((0, 1), (0, 5)))
    w_pad = jnp.pad(weights, ((0, 1), (0, 0), (0, 0)))

    nx_flat = _sc_gather_features(idx1d, ftab)
    px, py, pz = _sc_gather_positions(idx1d, ptab)
    px = px.reshape(N, K)
    py = py.reshape(N, K)
    pz = pz.reshape(N, K)

    out = pl.pallas_call(
        _tc_body,
        grid=(GRID,),
        in_specs=[
            pl.BlockSpec((B * K, D), lambda i: (i, 0)),
            pl.BlockSpec((B, K), lambda i: (i, 0)),
            pl.BlockSpec((B, K), lambda i: (i, 0)),
            pl.BlockSpec((B, K), lambda i: (i, 0)),
            pl.BlockSpec((B, 3), lambda i: (i, 0)),
            pl.BlockSpec(memory_space=pltpu.SMEM),
            pl.BlockSpec((P, D, OUT), lambda i: (0, 0, 0)),
        ],
        out_specs=pl.BlockSpec((B, OUT), lambda i: (i, 0)),
        out_shape=jax.ShapeDtypeStruct((N, OUT), jnp.float32),
        compiler_params=pltpu.CompilerParams(
            dimension_semantics=("arbitrary",),
        ),
    )(nx_flat, px, py, pz, q_pts, kp_pad, w_pad)
    return out


# speedup vs baseline: 6.2209x; 6.2209x over previous
import functools

import jax
import jax.numpy as jnp
from jax import lax
from jax.experimental import pallas as pl
from jax.experimental.pallas import tpu as pltpu
from jax.experimental.pallas import tpu_sc as plsc

N = 10000
K = 32
D = 128
OUT = 128
P = 16
KP_EXTENT = 0.05

E = N * K
IDX_ROWS = E // 128
NW = 32
ROWS_PER_W = IDX_ROWS // NW
TAIL_ROWS = IDX_ROWS - NW * ROWS_PER_W
GR = 3
GROUPS = ROWS_PER_W // GR
CHUNK = GR * 128

B = 200
GRID = N // B


def _sc_mesh():
    return plsc.VectorSubcoreMesh(
        core_axis_name="c", subcore_axis_name="s", num_cores=2, num_subcores=16
    )


def _worker_loop(body):
    wid = lax.axis_index("s") * 2 + lax.axis_index("c")
    wbase = wid * ROWS_PER_W

    def g_body(g, carry):
        body(wbase + g * GR, GR)
        return carry

    lax.fori_loop(0, GROUPS, g_body, 0)

    @pl.when(wid < TAIL_ROWS)
    def _tail():
        body(NW * ROWS_PER_W + wid, 1)


def _sc_gather_features(idx1d, ftab):
    @functools.partial(
        pl.kernel,
        out_type=jax.ShapeDtypeStruct((E, D), jnp.float32),
        mesh=_sc_mesh(),
        scratch_types=[
            pltpu.VMEM((CHUNK,), jnp.int32),
            pltpu.VMEM((CHUNK, D), jnp.float32),
            pltpu.SemaphoreType.DMA,
        ],
    )
    def k(idx_hbm, ftab_hbm, nx_hbm, idx_v, rows_v, sem):
        def do_chunk(r0, nrows):
            e0 = r0 * 128
            nedge = nrows * 128
            pltpu.sync_copy(idx_hbm.at[pl.ds(e0, nedge)],
                            idx_v.at[pl.ds(0, nedge)])
            descs = []
            for j in range(nrows):
                descs.append(pltpu.async_copy(
                    ftab_hbm.at[idx_v.at[pl.ds(j * 128, 128)]],
                    rows_v.at[pl.ds(j * 128, 128)], sem))
            for dsc in descs:
                dsc.wait()
            pltpu.sync_copy(rows_v.at[pl.ds(0, nedge)],
                            nx_hbm.at[pl.ds(e0, nedge)])

        _worker_loop(do_chunk)

    return k(idx1d, ftab)


def _sc_gather_positions(idx1d, ptab):
    @functools.partial(
        pl.kernel,
        out_type=(
            jax.ShapeDtypeStruct((E,), jnp.float32),
            jax.ShapeDtypeStruct((E,), jnp.float32),
            jax.ShapeDtypeStruct((E,), jnp.float32),
        ),
        mesh=_sc_mesh(),
        scratch_types=[
            pltpu.VMEM((CHUNK,), jnp.int32),
            pltpu.VMEM((CHUNK, 16), jnp.float32),
            pltpu.VMEM((CHUNK,), jnp.float32),
            pltpu.VMEM((CHUNK,), jnp.float32),
            pltpu.VMEM((CHUNK,), jnp.float32),
            pltpu.SemaphoreType.DMA,
        ],
        compiler_params=pltpu.CompilerParams(
            needs_layout_passes=False,
            use_tc_tiling_on_sc=False,
        ),
    )
    def k(idx_hbm, ptab_hbm, px_hbm, py_hbm, pz_hbm,
          idx_v, prow_v, pcx_v, pcy_v, pcz_v, sem):
        pc = (pcx_v, pcy_v, pcz_v)
        phbm = (px_hbm, py_hbm, pz_hbm)

        def do_chunk(r0, nrows):
            e0 = r0 * 128
            nedge = nrows * 128
            pltpu.sync_copy(idx_hbm.at[pl.ds(e0, nedge)],
                            idx_v.at[pl.ds(0, nedge)])
            descs = []
            for j in range(nrows):
                descs.append(pltpu.async_copy(
                    ptab_hbm.at[idx_v.at[pl.ds(j * 128, 128)]],
                    prow_v.at[pl.ds(j * 128, 128)], sem))
            for dsc in descs:
                dsc.wait()
            base_iota = lax.iota(jnp.int32, 16)

            def tr_body(g, carry):
                rows = g * 16 + base_iota
                for c in range(3):
                    v = plsc.load_gather(
                        prow_v, [rows, jnp.full((16,), c, jnp.int32)])
                    pc[c][pl.ds(g * 16, 16)] = v
                return carry

            lax.fori_loop(0, nedge // 16, tr_body, 0)
            for c in range(3):
                pltpu.sync_copy(pc[c].at[pl.ds(0, nedge)],
                                phbm[c].at[pl.ds(e0, nedge)])

        _worker_loop(do_chunk)

    return k(idx1d, ptab)


BM = 8
MB = B // BM
EM = BM * K


def _tc_body(nx_ref, px_ref, py_ref, pz_ref, q_ref, kp_ref, w_ref, o_ref,
             wf_ref, cnt_ref):
    lane = jax.lax.broadcasted_iota(jnp.int32, (BM, EM), 1)
    row = jax.lax.broadcasted_iota(jnp.int32, (BM, EM), 0)
    mask = (lane // K == row).astype(jnp.float32)
    maskb = mask.astype(jnp.bfloat16)

    for m in range(MB):
        xm = nx_ref[pl.ds(m * EM, EM), :]
        xmb = xm.astype(jnp.bfloat16)
        pxm = px_ref[pl.ds(m * BM, BM), :]
        pym = py_ref[pl.ds(m * BM, BM), :]
        pzm = pz_ref[pl.ds(m * BM, BM), :]
        dx = pxm - q_ref[pl.ds(m * BM, BM), 0:1]
        dy = pym - q_ref[pl.ds(m * BM, BM), 1:2]
        dz = pzm - q_ref[pl.ds(m * BM, BM), 2:3]

        s = jnp.sum(xm, axis=1, keepdims=True)
        ind = (s > 0.0).astype(jnp.float32)
        cnt = jnp.dot(mask, ind, preferred_element_type=jnp.float32)
        cnt_ref[pl.ds(m * BM, BM), :] = jnp.maximum(cnt, 1.0)

        for p in range(P):
            ex = dx - kp_ref[p, 0]
            ey = dy - kp_ref[p, 1]
            ez = dz - kp_ref[p, 2]
            sq = ex * ex + ey * ey + ez * ez
            w = jnp.maximum(1.0 - jnp.sqrt(sq) * (1.0 / KP_EXTENT), 0.0)
            a = jnp.tile(w, (1, BM)).astype(jnp.bfloat16) * maskb
            wf_ref[pl.ds(m * BM, BM), pl.ds(p * D, D)] = jnp.dot(
                a, xmb, preferred_element_type=jnp.float32)

    acc = jnp.dot(wf_ref[...], w_ref[...],
                  preferred_element_type=jnp.float32)
    o_ref[...] = acc * (1.0 / cnt_ref[...])


def kernel(q_pts, s_pts, x, neighbor_idxs, kernel_points, weights):
    idx1d = neighbor_idxs.astype(jnp.int32).reshape(E)
    ftab = jnp.concatenate([x, jnp.zeros((1, D), jnp.float32)], axis=0)
    ptab = jnp.pad(s_pts, ((0, 1), (0, 13)))
    kp_pad = jnp.pad(kernel_points, ((0, 1), (0, 5)))
    w_pad = jnp.pad(weights, ((0, 1), (0, 0), (0, 0)))

    nx_flat = _sc_gather_features(idx1d, ftab)
    px, py, pz = _sc_gather_positions(idx1d, ptab)
    px = px.reshape(N, K)
    py = py.reshape(N, K)
    pz = pz.reshape(N, K)

    out = pl.pallas_call(
        _tc_body,
        grid=(GRID,),
        in_specs=[
            pl.BlockSpec((B * K, D), lambda i: (i, 0)),
            pl.BlockSpec((B, K), lambda i: (i, 0)),
            pl.BlockSpec((B, K), lambda i: (i, 0)),
            pl.BlockSpec((B, K), lambda i: (i, 0)),
            pl.BlockSpec((B, 3), lambda i: (i, 0)),
            pl.BlockSpec(memory_space=pltpu.SMEM),
            pl.BlockSpec((P * D, OUT), lambda i: (0, 0)),
        ],
        out_specs=pl.BlockSpec((B, OUT), lambda i: (i, 0)),
        out_shape=jax.ShapeDtypeStruct((N, OUT), jnp.float32),
        scratch_shapes=[
            pltpu.VMEM((B, P * D), jnp.float32),
            pltpu.VMEM((B, 1), jnp.float32),
        ],
        compiler_params=pltpu.CompilerParams(
            dimension_semantics=("arbitrary",),
        ),
    )(nx_flat, px, py, pz, q_pts, kp_pad, w_pad.reshape(P * D, OUT))
    return out


# speedup vs baseline: 6.7856x; 1.0908x over previous
import functools

import jax
import jax.numpy as jnp
from jax import lax
from jax.experimental import pallas as pl
from jax.experimental.pallas import tpu as pltpu
from jax.experimental.pallas import tpu_sc as plsc

N = 10000
K = 32
D = 128
OUT = 128
P = 16
KP_EXTENT = 0.05

E = N * K
IDX_ROWS = E // 128
NW = 32
ROWS_PER_W = IDX_ROWS // NW
TAIL_ROWS = IDX_ROWS - NW * ROWS_PER_W
GR = 3
GROUPS = ROWS_PER_W // GR
CHUNK = GR * 128

B = 200
GRID = N // B


def _sc_mesh():
    return plsc.VectorSubcoreMesh(
        core_axis_name="c", subcore_axis_name="s", num_cores=2, num_subcores=16
    )


def _sc_gather_features(idx1d, ftab):
    @functools.partial(
        pl.kernel,
        out_type=jax.ShapeDtypeStruct((E, D), jnp.float32),
        mesh=_sc_mesh(),
        scratch_types=[
            pltpu.VMEM((CHUNK,), jnp.int32),
            pltpu.VMEM((CHUNK,), jnp.int32),
            pltpu.VMEM((CHUNK, D), jnp.float32),
            pltpu.VMEM((CHUNK, D), jnp.float32),
            pltpu.SemaphoreType.DMA,
            pltpu.SemaphoreType.DMA,
            pltpu.SemaphoreType.DMA,
            pltpu.SemaphoreType.DMA,
            pltpu.SemaphoreType.DMA,
            pltpu.SemaphoreType.DMA,
        ],
    )
    def k(idx_hbm, ftab_hbm, nx_hbm, idx0, idx1, rows0, rows1,
          si0, si1, sg0, sg1, sw0, sw1):
        wid = lax.axis_index("s") * 2 + lax.axis_index("c")
        wbase = wid * ROWS_PER_W
        idxs, rows = (idx0, idx1), (rows0, rows1)
        sis, sgs, sws = (si0, si1), (sg0, sg1), (sw0, sw1)

        def fire_idx(g, b):
            pltpu.async_copy(
                idx_hbm.at[pl.ds((wbase + g * GR) * 128, CHUNK)],
                idxs[b], sis[b])

        def wait_idx(b):
            pltpu.make_async_copy(
                idx_hbm.at[pl.ds(0, CHUNK)], idxs[b], sis[b]).wait()

        def fire_gather(b):
            for j in range(GR):
                pltpu.async_copy(
                    ftab_hbm.at[idxs[b].at[pl.ds(j * 128, 128)]],
                    rows[b].at[pl.ds(j * 128, 128)], sgs[b])

        def wait_gather(b):
            for j in range(GR):
                pltpu.make_async_copy(
                    ftab_hbm.at[idxs[b].at[pl.ds(j * 128, 128)]],
                    rows[b].at[pl.ds(j * 128, 128)], sgs[b]).wait()

        def fire_write(g, b):
            pltpu.async_copy(
                rows[b], nx_hbm.at[pl.ds((wbase + g * GR) * 128, CHUNK)],
                sws[b])

        def wait_write(b):
            pltpu.make_async_copy(
                rows[b], nx_hbm.at[pl.ds(0, CHUNK)], sws[b]).wait()

        fire_idx(0, 0)

        def pair(gp, c):
            for b in (0, 1):
                g = gp * 2 + b

                @pl.when(g + 1 < GROUPS)
                def _():
                    fire_idx(g + 1, 1 - b)

                wait_idx(b)

                @pl.when(gp >= 1)
                def _():
                    wait_write(b)

                fire_gather(b)
                wait_gather(b)
                fire_write(g, b)
            return c

        lax.fori_loop(0, GROUPS // 2, pair, 0)
        wait_write(0)
        wait_write(1)

        @pl.when(wid < TAIL_ROWS)
        def _tail():
            r0 = NW * ROWS_PER_W + wid
            pltpu.async_copy(idx_hbm.at[pl.ds(r0 * 128, 128)],
                             idx0.at[pl.ds(0, 128)], si0).wait()
            pltpu.async_copy(ftab_hbm.at[idx0.at[pl.ds(0, 128)]],
                             rows0.at[pl.ds(0, 128)], sg0).wait()
            pltpu.async_copy(rows0.at[pl.ds(0, 128)],
                             nx_hbm.at[pl.ds(r0 * 128, 128)], sw0).wait()

    return k(idx1d, ftab)


def _sc_gather_positions(idx1d, ptab):
    @functools.partial(
        pl.kernel,
        out_type=(
            jax.ShapeDtypeStruct((E,), jnp.float32),
            jax.ShapeDtypeStruct((E,), jnp.float32),
            jax.ShapeDtypeStruct((E,), jnp.float32),
        ),
        mesh=_sc_mesh(),
        scratch_types=[
            pltpu.VMEM((CHUNK,), jnp.int32),
            pltpu.VMEM((CHUNK,), jnp.int32),
            pltpu.VMEM((CHUNK, 16), jnp.float32),
            pltpu.VMEM((CHUNK, 16), jnp.float32),
            pltpu.VMEM((3, CHUNK), jnp.float32),
            pltpu.VMEM((3, CHUNK), jnp.float32),
            pltpu.SemaphoreType.DMA,
            pltpu.SemaphoreType.DMA,
            pltpu.SemaphoreType.DMA,
            pltpu.SemaphoreType.DMA,
            pltpu.SemaphoreType.DMA,
            pltpu.SemaphoreType.DMA,
        ],
        compiler_params=pltpu.CompilerParams(
            needs_layout_passes=False,
            use_tc_tiling_on_sc=False,
        ),
    )
    def k(idx_hbm, ptab_hbm, px_hbm, py_hbm, pz_hbm,
          idx0, idx1, prow0, prow1, pc0, pc1,
          si0, si1, sg0, sg1, sw0, sw1):
        wid = lax.axis_index("s") * 2 + lax.axis_index("c")
        wbase = wid * ROWS_PER_W
        idxs, prows, pcs = (idx0, idx1), (prow0, prow1), (pc0, pc1)
        sis, sgs, sws = (si0, si1), (sg0, sg1), (sw0, sw1)
        phbm = (px_hbm, py_hbm, pz_hbm)
        base_iota = lax.iota(jnp.int32, 16)

        def fire_idx(g, b):
            pltpu.async_copy(
                idx_hbm.at[pl.ds((wbase + g * GR) * 128, CHUNK)],
                idxs[b], sis[b])

        def wait_idx(b):
            pltpu.make_async_copy(
                idx_hbm.at[pl.ds(0, CHUNK)], idxs[b], sis[b]).wait()

        def fire_gather(b):
            for j in range(GR):
                pltpu.async_copy(
                    ptab_hbm.at[idxs[b].at[pl.ds(j * 128, 128)]],
                    prows[b].at[pl.ds(j * 128, 128)], sgs[b])

        def wait_gather(b):
            for j in range(GR):
                pltpu.make_async_copy(
                    ptab_hbm.at[idxs[b].at[pl.ds(j * 128, 128)]],
                    prows[b].at[pl.ds(j * 128, 128)], sgs[b]).wait()

        def transpose(b, nedge):
            def tr_body(g, carry):
                rows = g * 16 + base_iota
                for c in range(3):
                    v = plsc.load_gather(
                        prows[b], [rows, jnp.full((16,), c, jnp.int32)])
                    pcs[b][c, pl.ds(g * 16, 16)] = v
                return carry

            lax.fori_loop(0, nedge // 16, tr_body, 0)

        def fire_write(g, b):
            e0 = (wbase + g * GR) * 128
            for c in range(3):
                pltpu.async_copy(pcs[b].at[c],
                                 phbm[c].at[pl.ds(e0, CHUNK)], sws[b])

        def wait_write(b):
            for c in range(3):
                pltpu.make_async_copy(
                    pcs[b].at[c], phbm[c].at[pl.ds(0, CHUNK)], sws[b]).wait()

        fire_idx(0, 0)

        def pair(gp, c):
            for b in (0, 1):
                g = gp * 2 + b

                @pl.when(g + 1 < GROUPS)
                def _():
                    fire_idx(g + 1, 1 - b)

                wait_idx(b)
                fire_gather(b)
                wait_gather(b)

                @pl.when(gp >= 1)
                def _():
                    wait_write(b)

                transpose(b, CHUNK)
                fire_write(g, b)
            return c

        lax.fori_loop(0, GROUPS // 2, pair, 0)
        wait_write(0)
        wait_write(1)

        @pl.when(wid < TAIL_ROWS)
        def _tail():
            r0 = NW * ROWS_PER_W + wid
            pltpu.async_copy(idx_hbm.at[pl.ds(r0 * 128, 128)],
                             idx0.at[pl.ds(0, 128)], si0).wait()
            pltpu.async_copy(ptab_hbm.at[idx0.at[pl.ds(0, 128)]],
                             prow0.at[pl.ds(0, 128)], sg0).wait()
            transpose(0, 128)
            for c in range(3):
                pltpu.async_copy(pc0.at[c, pl.ds(0, 128)],
                                 phbm[c].at[pl.ds(r0 * 128, 128)], sw0).wait()

    return k(idx1d, ptab)


BM = 8
MB = B // BM
EM = BM * K


def _tc_body(nx_ref, px_ref, py_ref, pz_ref, q_ref, kp_ref, w_ref, o_ref,
             wf_ref, cnt_ref):
    lane = jax.lax.broadcasted_iota(jnp.int32, (BM, EM), 1)
    row = jax.lax.broadcasted_iota(jnp.int32, (BM, EM), 0)
    mask = (lane // K == row).astype(jnp.float32)
    maskb = mask.astype(jnp.bfloat16)

    for m in range(MB):
        xm = nx_ref[pl.ds(m * EM, EM), :]
        xmb = xm.astype(jnp.bfloat16)
        pxm = px_ref[pl.ds(m * BM, BM), :]
        pym = py_ref[pl.ds(m * BM, BM), :]
        pzm = pz_ref[pl.ds(m * BM, BM), :]
        dx = pxm - q_ref[pl.ds(m * BM, BM), 0:1]
        dy = pym - q_ref[pl.ds(m * BM, BM), 1:2]
        dz = pzm - q_ref[pl.ds(m * BM, BM), 2:3]

        s = jnp.sum(xm, axis=1, keepdims=True)
        ind = (s > 0.0).astype(jnp.float32)
        cnt = jnp.dot(mask, ind, preferred_element_type=jnp.float32)
        cnt_ref[pl.ds(m * BM, BM), :] = jnp.maximum(cnt, 1.0)

        for p in range(P):
            ex = dx - kp_ref[p, 0]
            ey = dy - kp_ref[p, 1]
            ez = dz - kp_ref[p, 2]
            sq = ex * ex + ey * ey + ez * ez
            w = jnp.maximum(1.0 - jnp.sqrt(sq) * (1.0 / KP_EXTENT), 0.0)
            a = jnp.tile(w, (1, BM)).astype(jnp.bfloat16) * maskb
            wf_ref[pl.ds(m * BM, BM), pl.ds(p * D, D)] = jnp.dot(
                a, xmb, preferred_element_type=jnp.float32)

    acc = jnp.dot(wf_ref[...], w_ref[...],
                  preferred_element_type=jnp.float32)
    o_ref[...] = acc * (1.0 / cnt_ref[...])


def kernel(q_pts, s_pts, x, neighbor_idxs, kernel_points, weights):
    idx1d = neighbor_idxs.astype(jnp.int32).reshape(E)
    ftab = jnp.concatenate([x, jnp.zeros((1, D), jnp.float32)], axis=0)
    ptab = jnp.pad(s_pts, ((0, 1), (0, 13)))
    kp_pad = jnp.pad(kernel_points, ((0, 1), (0, 5)))
    w_pad = jnp.pad(weights, ((0, 1), (0, 0), (0, 0)))

    nx_flat = _sc_gather_features(idx1d, ftab)
    px, py, pz = _sc_gather_positions(idx1d, ptab)
    px = px.reshape(N, K)
    py = py.reshape(N, K)
    pz = pz.reshape(N, K)

    out = pl.pallas_call(
        _tc_body,
        grid=(GRID,),
        in_specs=[
            pl.BlockSpec((B * K, D), lambda i: (i, 0)),
            pl.BlockSpec((B, K), lambda i: (i, 0)),
            pl.BlockSpec((B, K), lambda i: (i, 0)),
            pl.BlockSpec((B, K), lambda i: (i, 0)),
            pl.BlockSpec((B, 3), lambda i: (i, 0)),
            pl.BlockSpec(memory_space=pltpu.SMEM),
            pl.BlockSpec((P * D, OUT), lambda i: (0, 0)),
        ],
        out_specs=pl.BlockSpec((B, OUT), lambda i: (i, 0)),
        out_shape=jax.ShapeDtypeStruct((N, OUT), jnp.float32),
        scratch_shapes=[
            pltpu.VMEM((B, P * D), jnp.float32),
            pltpu.VMEM((B, 1), jnp.float32),
        ],
        compiler_params=pltpu.CompilerParams(
            dimension_semantics=("arbitrary",),
        ),
    )(nx_flat, px, py, pz, q_pts, kp_pad, w_pad.reshape(P * D, OUT))
    return out
